# baseline (speedup 1.0000x reference)
import jax
import jax.numpy as jnp
from jax import lax
from jax.experimental import pallas as pl
from jax.experimental.pallas import tpu as pltpu
from jax.experimental.pallas import tpu_sc as plsc

VOCAB = 1000000
D = 32
BATCH = 4096
HIST = 50
LANES = 16

NUM_CORES = 2
NUM_SUBCORES = 16
NW = NUM_CORES * NUM_SUBCORES
ROWS_PER_W = BATCH // NW
CHUNK_ROWS = 16
NCHUNKS = ROWS_PER_W // CHUNK_ROWS
CHUNK_PAIRS = CHUNK_ROWS * HIST
WPAD = 64

_BCAST_DNUMS = lax.GatherDimensionNumbers(
    offset_dims=(), collapsed_slice_dims=(0,), start_index_map=(0,))


def _lane_bcast(vec, k):
    idx = jnp.full((LANES, 1), k, jnp.int32)
    return lax.gather(vec, idx, _BCAST_DNUMS, (1,),
                      mode=lax.GatherScatterMode.PROMISE_IN_BOUNDS)


def _nbow_kernel(idx_hbm, table_hbm, tw_hbm, out_hbm,
                 idx_v, rows_v, w_v, w_pad, out_v, sem_w, sem_r):
    wid = lax.axis_index("s") * NUM_CORES + lax.axis_index("c")

    for i in range(CHUNK_ROWS):
        w_pad[pl.ds(i * WPAD + 48, LANES)] = jnp.zeros((LANES,), jnp.float32)

    def chunk_body(c, carry):
        base_pair = (wid * ROWS_PER_W + c * CHUNK_ROWS) * HIST
        base_row = wid * ROWS_PER_W + c * CHUNK_ROWS

        pltpu.sync_copy(idx_hbm.at[pl.ds(base_pair, CHUNK_PAIRS)], idx_v)
        cp_w = pltpu.async_copy(tw_hbm.at[idx_v], w_v, sem_w)
        cp_r = pltpu.async_copy(table_hbm.at[idx_v], rows_v, sem_r)
        cp_w.wait()
        cp_r.wait()

        def mask_body(g, carry2):
            pv = g * LANES + lax.iota(jnp.int32, LANES)
            iv = idx_v[pl.ds(g * LANES, LANES)]
            tw16 = w_v[pl.ds(g * LANES, LANES)]
            w = jnp.where(iv != 0, tw16, 0.0)
            hv = jnp.full((LANES,), HIST, jnp.int32)
            dest = lax.div(pv, hv) * WPAD + lax.rem(pv, hv)
            plsc.store_scatter(w_pad, [dest], w)
            return carry2
        lax.fori_loop(0, CHUNK_PAIRS // LANES, mask_body, 0, unroll=4)

        def row_body(i, carry2):
            p0 = i * HIST
            acc0 = jnp.zeros((LANES,), jnp.float32)
            acc1 = jnp.zeros((LANES,), jnp.float32)
            for g in range(4):
                w16 = w_pad[pl.ds(i * WPAD + g * LANES, LANES)]
                for k in range(LANES if g < 3 else HIST - 3 * LANES):
                    p = p0 + g * LANES + k
                    wv = _lane_bcast(w16, k)
                    acc0 = acc0 + wv * rows_v[p, 0:LANES]
                    acc1 = acc1 + wv * rows_v[p, LANES:2 * LANES]
            out_v[i, 0:LANES] = acc0
            out_v[i, LANES:2 * LANES] = acc1
            return carry2
        lax.fori_loop(0, CHUNK_ROWS, row_body, 0)

        pltpu.sync_copy(out_v, out_hbm.at[pl.ds(base_row, CHUNK_ROWS)])
        return carry

    lax.fori_loop(0, NCHUNKS, chunk_body, 0)


@jax.jit
def kernel(idxs, table, token_weights):
    idx_flat = idxs.reshape(BATCH * HIST).astype(jnp.int32)
    mesh = plsc.VectorSubcoreMesh(core_axis_name="c", subcore_axis_name="s")
    f = pl.kernel(
        _nbow_kernel,
        mesh=mesh,
        compiler_params=pltpu.CompilerParams(
            use_tc_tiling_on_sc=False, needs_layout_passes=False),
        out_type=jax.ShapeDtypeStruct((BATCH, D), jnp.float32),
        scratch_types=[
            pltpu.VMEM((CHUNK_PAIRS,), jnp.int32),
            pltpu.VMEM((CHUNK_PAIRS, D), jnp.float32),
            pltpu.VMEM((CHUNK_PAIRS,), jnp.float32),
            pltpu.VMEM((CHUNK_ROWS * WPAD,), jnp.float32),
            pltpu.VMEM((CHUNK_ROWS, D), jnp.float32),
            pltpu.SemaphoreType.DMA,
            pltpu.SemaphoreType.DMA,
        ],
    )
    return f(idx_flat, table, token_weights)
